# baseline (speedup 1.0000x reference)
import functools

import jax
import jax.numpy as jnp
from jax import lax
from jax.experimental import pallas as pl
from jax.experimental.pallas import tpu as pltpu

_M = 400
_K = 10
_Q = 256
_N_DB = 100000
_D = 512
_QB = 4

_HI = lax.Precision.HIGHEST


def _stable_desc_rank(v_row, v_col, tie_mask):
    m = v_row.shape[1]
    r_mat = jnp.broadcast_to(v_row, (m, m))
    l_mat = jnp.broadcast_to(v_col, (m, m))
    gt = r_mat > l_mat
    tie = (r_mat == l_mat) & tie_mask
    return jnp.sum((gt | tie).astype(jnp.float32), axis=1, keepdims=True)


def _onehot_collect(rank_col, data_col, iota_lane_f, m):
    oh = (jnp.broadcast_to(rank_col, (m, m)) == iota_lane_f)
    contrib = jnp.where(oh, jnp.broadcast_to(data_col, (m, m)), 0.0)
    return jnp.sum(contrib, axis=0, keepdims=True)


def _compute_body(res_ref, perm_ref, val_ref, idx_ref, x_ref, out_ref):
    m = _M
    iota_lane = lax.broadcasted_iota(jnp.int32, (m, m), 1)
    iota_sub = lax.broadcasted_iota(jnp.int32, (m, m), 0)
    ident = (iota_lane == iota_sub).astype(jnp.float32)
    iota_lane_f = iota_lane.astype(jnp.float32)
    tie_mask = iota_lane < iota_sub

    for j in range(_QB):
        v_row = res_ref[0, j:j + 1, :]
        val_row = val_ref[0, j:j + 1, :].astype(jnp.float32)
        pair = jnp.concatenate([v_row, val_row], axis=0)
        pair_col = lax.dot_general(ident, pair, (((1,), (1,)), ((), ())),
                                   precision=_HI)
        v_col = pair_col[:, 0:1]
        val_col = pair_col[:, 1:2]

        rank1 = _stable_desc_rank(v_row, v_col, tie_mask)
        sv_row = _onehot_collect(rank1, v_col, iota_lane_f, m)

        x_mat = x_ref[0, j]
        vmax = x_ref[0, j, pl.ds(idx_ref[0, j, 0], 1), :]
        for k in range(1, _K):
            vmax = jnp.maximum(vmax, x_ref[0, j, pl.ds(idx_ref[0, j, k], 1), :])

        s_row = lax.dot_general(vmax, x_mat, (((1,), (1,)), ((), ())),
                                precision=lax.Precision.DEFAULT)
        s_col = lax.dot_general(ident, s_row, (((1,), (1,)), ((), ())),
                                precision=_HI)

        perm_row = perm_ref[0, j:j + 1, :]
        p_mask = (iota_sub == jnp.broadcast_to(perm_row, (m, m)))
        s_perm = jnp.sum(
            jnp.where(p_mask, jnp.broadcast_to(s_col, (m, m)), 0.0),
            axis=0, keepdims=True)

        r_row = (sv_row + s_perm) * 0.5
        r_col = lax.dot_general(ident, r_row, (((1,), (1,)), ((), ())),
                                precision=_HI)
        rank2 = _stable_desc_rank(r_row, r_col, tie_mask)

        reord = _onehot_collect(rank2, val_col, iota_lane_f, m)
        out_ref[0, j:j + 1, :] = reord.astype(jnp.int32)


def _assembly_body(reord_ref, ranks_ref, out_ref, *, rows):
    i = pl.program_id(0)

    @pl.when(i == 0)
    def _():
        rf = reord_ref[...].astype(jnp.float32)
        il = lax.broadcasted_iota(jnp.int32, (_Q, _Q), 1)
        isub = lax.broadcasted_iota(jnp.int32, (_Q, _Q), 0)
        ident_q = (il == isub).astype(jnp.float32)
        t = lax.dot_general(rf, ident_q, (((0,), (0,)), ((), ())),
                            precision=_HI)
        out_ref[0:_M, :] = t.astype(jnp.int32)
        out_ref[_M:rows, :] = ranks_ref[_M:rows, :]

    @pl.when(i != 0)
    def _():
        out_ref[...] = ranks_ref[...]


def kernel(ranks, rerank_dba_final, res_top1000_dba, ranks_trans_1000_pre, x_dba):
    q, m, d, qb = _Q, _M, _D, _QB
    ng = q // qb

    res_b = res_top1000_dba.reshape(ng, qb, m)
    perm_b = ranks_trans_1000_pre.reshape(ng, qb, m)
    val_b = rerank_dba_final.reshape(ng, qb, m)
    idx_top = ranks_trans_1000_pre[:, :_K].reshape(ng, qb, _K)
    x_b = x_dba.reshape(ng, qb, m, d)

    reordered = pl.pallas_call(
        _compute_body,
        grid=(ng,),
        in_specs=[
            pl.BlockSpec((1, qb, m), lambda i: (i, 0, 0)),
            pl.BlockSpec((1, qb, m), lambda i: (i, 0, 0)),
            pl.BlockSpec((1, qb, m), lambda i: (i, 0, 0)),
            pl.BlockSpec((1, qb, _K), lambda i: (i, 0, 0),
                         memory_space=pltpu.SMEM),
            pl.BlockSpec((1, qb, m, d), lambda i: (i, 0, 0, 0)),
        ],
        out_specs=pl.BlockSpec((1, qb, m), lambda i: (i, 0, 0)),
        out_shape=jax.ShapeDtypeStruct((ng, qb, m), jnp.int32),
    )(res_b, perm_b, val_b, idx_top, x_b)

    rows = 4000
    nblk = _N_DB // rows
    out = pl.pallas_call(
        functools.partial(_assembly_body, rows=rows),
        grid=(nblk,),
        in_specs=[
            pl.BlockSpec((q, m), lambda i: (0, 0)),
            pl.BlockSpec((rows, q), lambda i: (i, 0)),
        ],
        out_specs=pl.BlockSpec((rows, q), lambda i: (i, 0)),
        out_shape=jax.ShapeDtypeStruct((_N_DB, q), jnp.int32),
    )(reordered.reshape(q, m), ranks)
    return out


# speedup vs baseline: 1.4738x; 1.4738x over previous
import functools

import jax
import jax.numpy as jnp
from jax import lax
from jax.experimental import pallas as pl
from jax.experimental.pallas import tpu as pltpu

_M = 400
_K = 10
_Q = 256
_N_DB = 100000
_D = 512
_QB = 4

_HI = lax.Precision.HIGHEST


def _stable_desc_rank(v_row, v_col, tie_mask):
    m = v_row.shape[1]
    r_mat = jnp.broadcast_to(v_row, (m, m))
    l_mat = jnp.broadcast_to(v_col, (m, m))
    gt = r_mat > l_mat
    tie = (r_mat == l_mat) & tie_mask
    return jnp.sum((gt | tie).astype(jnp.float32), axis=1, keepdims=True)


def _onehot_collect(rank_col, data_col, iota_lane_f, m):
    oh = (jnp.broadcast_to(rank_col, (m, m)) == iota_lane_f)
    contrib = jnp.where(oh, jnp.broadcast_to(data_col, (m, m)), 0.0)
    return jnp.sum(contrib, axis=0, keepdims=True)


def _compute_body(res_ref, perm_ref, val_ref, idx_ref, x_ref, out_ref):
    m = _M
    iota_lane = lax.broadcasted_iota(jnp.int32, (m, m), 1)
    iota_sub = lax.broadcasted_iota(jnp.int32, (m, m), 0)
    ident = (iota_lane == iota_sub).astype(jnp.float32)
    iota_lane_f = iota_lane.astype(jnp.float32)
    tie_mask = iota_lane < iota_sub

    for j in range(_QB):
        v_row = res_ref[0, j:j + 1, :]
        val_row = val_ref[0, j:j + 1, :].astype(jnp.float32)
        v_col = lax.dot_general(ident, v_row, (((1,), (1,)), ((), ())),
                                precision=_HI)
        val_col = lax.dot_general(ident, val_row, (((1,), (1,)), ((), ())),
                                  precision=_HI)

        rank1 = _stable_desc_rank(v_row, v_col, tie_mask)
        sv_row = _onehot_collect(rank1, v_col, iota_lane_f, m)

        x_mat = x_ref[0, j]
        vmax = x_ref[0, j, pl.ds(idx_ref[0, j, 0], 1), :]
        for k in range(1, _K):
            vmax = jnp.maximum(vmax, x_ref[0, j, pl.ds(idx_ref[0, j, k], 1), :])

        s_row = lax.dot_general(vmax, x_mat, (((1,), (1,)), ((), ())),
                                precision=lax.Precision.DEFAULT)
        s_col = lax.dot_general(ident, s_row, (((1,), (1,)), ((), ())),
                                precision=_HI)

        perm_row = perm_ref[0, j:j + 1, :]
        p_mask = (iota_sub == jnp.broadcast_to(perm_row, (m, m)))
        s_perm = jnp.sum(
            jnp.where(p_mask, jnp.broadcast_to(s_col, (m, m)), 0.0),
            axis=0, keepdims=True)

        r_row = (sv_row + s_perm) * 0.5
        r_col = lax.dot_general(ident, r_row, (((1,), (1,)), ((), ())),
                                precision=_HI)
        rank2 = _stable_desc_rank(r_row, r_col, tie_mask)

        reord = _onehot_collect(rank2, val_col, iota_lane_f, m)
        out_ref[0, j:j + 1, :] = reord.astype(jnp.int32)


def _assembly_body(reord_ref, ranks_ref, out_ref, *, rows):
    i = pl.program_id(0)

    @pl.when(i == 0)
    def _():
        rf = reord_ref[...].astype(jnp.float32)
        il = lax.broadcasted_iota(jnp.int32, (_Q, _Q), 1)
        isub = lax.broadcasted_iota(jnp.int32, (_Q, _Q), 0)
        ident_q = (il == isub).astype(jnp.float32)
        t = lax.dot_general(rf, ident_q, (((0,), (0,)), ((), ())),
                            precision=_HI)
        out_ref[0:_M, :] = t.astype(jnp.int32)
        out_ref[_M:rows, :] = ranks_ref[_M:rows, :]

    @pl.when(i != 0)
    def _():
        out_ref[...] = ranks_ref[...]


def kernel(ranks, rerank_dba_final, res_top1000_dba, ranks_trans_1000_pre, x_dba):
    q, m, d, qb = _Q, _M, _D, _QB
    ng = q // qb

    res_b = res_top1000_dba.reshape(ng, qb, m)
    perm_b = ranks_trans_1000_pre.reshape(ng, qb, m)
    val_b = rerank_dba_final.reshape(ng, qb, m)
    idx_top = ranks_trans_1000_pre[:, :_K].reshape(ng, qb, _K)
    x_b = x_dba.reshape(ng, qb, m, d)

    reordered = pl.pallas_call(
        _compute_body,
        grid=(ng,),
        in_specs=[
            pl.BlockSpec((1, qb, m), lambda i: (i, 0, 0)),
            pl.BlockSpec((1, qb, m), lambda i: (i, 0, 0)),
            pl.BlockSpec((1, qb, m), lambda i: (i, 0, 0)),
            pl.BlockSpec((1, qb, _K), lambda i: (i, 0, 0),
                         memory_space=pltpu.SMEM),
            pl.BlockSpec((1, qb, m, d), lambda i: (i, 0, 0, 0)),
        ],
        out_specs=pl.BlockSpec((1, qb, m), lambda i: (i, 0, 0)),
        out_shape=jax.ShapeDtypeStruct((ng, qb, m), jnp.int32),
    )(res_b, perm_b, val_b, idx_top, x_b)

    rows = 4000
    nblk = _N_DB // rows
    out = pl.pallas_call(
        functools.partial(_assembly_body, rows=rows),
        grid=(nblk,),
        in_specs=[
            pl.BlockSpec((q, m), lambda i: (0, 0)),
            pl.BlockSpec((rows, q), lambda i: (i, 0)),
        ],
        out_specs=pl.BlockSpec((rows, q), lambda i: (i, 0)),
        out_shape=jax.ShapeDtypeStruct((_N_DB, q), jnp.int32),
    )(reordered.reshape(q, m), ranks)
    return out


# speedup vs baseline: 1.4857x; 1.0081x over previous
import functools

import jax
import jax.numpy as jnp
from jax import lax
from jax.experimental import pallas as pl
from jax.experimental.pallas import tpu as pltpu

_M = 400
_K = 10
_Q = 256
_N_DB = 100000
_D = 512
_QB = 8

_HI = lax.Precision.HIGHEST


def _stable_desc_rank(v_row, v_col, tie_mask):
    m = v_row.shape[1]
    r_mat = jnp.broadcast_to(v_row, (m, m))
    l_mat = jnp.broadcast_to(v_col, (m, m))
    gt = r_mat > l_mat
    tie = (r_mat == l_mat) & tie_mask
    return jnp.sum((gt | tie).astype(jnp.float32), axis=1, keepdims=True)


def _onehot_collect(rank_col, data_col, iota_lane_f, m):
    oh = (jnp.broadcast_to(rank_col, (m, m)) == iota_lane_f)
    contrib = jnp.where(oh, jnp.broadcast_to(data_col, (m, m)), 0.0)
    return jnp.sum(contrib, axis=0, keepdims=True)


def _compute_body(res_ref, perm_ref, val_ref, idx_ref, x_ref, out_ref):
    m = _M
    iota_lane = lax.broadcasted_iota(jnp.int32, (m, m), 1)
    iota_sub = lax.broadcasted_iota(jnp.int32, (m, m), 0)
    ident = (iota_lane == iota_sub).astype(jnp.float32)
    iota_lane_f = iota_lane.astype(jnp.float32)
    tie_mask = iota_lane < iota_sub

    for j in range(_QB):
        v_row = res_ref[0, j:j + 1, :]
        val_row = val_ref[0, j:j + 1, :].astype(jnp.float32)
        v_col = lax.dot_general(ident, v_row, (((1,), (1,)), ((), ())),
                                precision=_HI)
        val_col = lax.dot_general(ident, val_row, (((1,), (1,)), ((), ())),
                                  precision=_HI)

        rank1 = _stable_desc_rank(v_row, v_col, tie_mask)
        sv_row = _onehot_collect(rank1, v_col, iota_lane_f, m)

        x_mat = x_ref[0, j]
        vmax = x_ref[0, j, pl.ds(idx_ref[0, j, 0], 1), :]
        for k in range(1, _K):
            vmax = jnp.maximum(vmax, x_ref[0, j, pl.ds(idx_ref[0, j, k], 1), :])

        s_row = lax.dot_general(vmax, x_mat, (((1,), (1,)), ((), ())),
                                precision=lax.Precision.DEFAULT)
        s_col = lax.dot_general(ident, s_row, (((1,), (1,)), ((), ())),
                                precision=_HI)

        perm_row = perm_ref[0, j:j + 1, :]
        p_mask = (iota_sub == jnp.broadcast_to(perm_row, (m, m)))
        s_perm = jnp.sum(
            jnp.where(p_mask, jnp.broadcast_to(s_col, (m, m)), 0.0),
            axis=0, keepdims=True)

        r_row = (sv_row + s_perm) * 0.5
        r_col = lax.dot_general(ident, r_row, (((1,), (1,)), ((), ())),
                                precision=_HI)
        rank2 = _stable_desc_rank(r_row, r_col, tie_mask)

        reord = _onehot_collect(rank2, val_col, iota_lane_f, m)
        out_ref[0, j:j + 1, :] = reord.astype(jnp.int32)


def _assembly_body(reord_ref, ranks_ref, out_ref, *, rows):
    i = pl.program_id(0)

    @pl.when(i == 0)
    def _():
        rf = reord_ref[...].astype(jnp.float32)
        il = lax.broadcasted_iota(jnp.int32, (_Q, _Q), 1)
        isub = lax.broadcasted_iota(jnp.int32, (_Q, _Q), 0)
        ident_q = (il == isub).astype(jnp.float32)
        t = lax.dot_general(rf, ident_q, (((0,), (0,)), ((), ())),
                            precision=_HI)
        out_ref[0:_M, :] = t.astype(jnp.int32)
        out_ref[_M:rows, :] = ranks_ref[_M:rows, :]

    @pl.when(i != 0)
    def _():
        out_ref[...] = ranks_ref[...]


def kernel(ranks, rerank_dba_final, res_top1000_dba, ranks_trans_1000_pre, x_dba):
    q, m, d, qb = _Q, _M, _D, _QB
    ng = q // qb

    res_b = res_top1000_dba.reshape(ng, qb, m)
    perm_b = ranks_trans_1000_pre.reshape(ng, qb, m)
    val_b = rerank_dba_final.reshape(ng, qb, m)
    idx_top = ranks_trans_1000_pre[:, :_K].reshape(ng, qb, _K)
    x_b = x_dba.reshape(ng, qb, m, d)

    reordered = pl.pallas_call(
        _compute_body,
        grid=(ng,),
        in_specs=[
            pl.BlockSpec((1, qb, m), lambda i: (i, 0, 0)),
            pl.BlockSpec((1, qb, m), lambda i: (i, 0, 0)),
            pl.BlockSpec((1, qb, m), lambda i: (i, 0, 0)),
            pl.BlockSpec((1, qb, _K), lambda i: (i, 0, 0),
                         memory_space=pltpu.SMEM),
            pl.BlockSpec((1, qb, m, d), lambda i: (i, 0, 0, 0)),
        ],
        out_specs=pl.BlockSpec((1, qb, m), lambda i: (i, 0, 0)),
        out_shape=jax.ShapeDtypeStruct((ng, qb, m), jnp.int32),
    )(res_b, perm_b, val_b, idx_top, x_b)

    rows = 4000
    nblk = _N_DB // rows
    out = pl.pallas_call(
        functools.partial(_assembly_body, rows=rows),
        grid=(nblk,),
        in_specs=[
            pl.BlockSpec((q, m), lambda i: (0, 0)),
            pl.BlockSpec((rows, q), lambda i: (i, 0)),
        ],
        out_specs=pl.BlockSpec((rows, q), lambda i: (i, 0)),
        out_shape=jax.ShapeDtypeStruct((_N_DB, q), jnp.int32),
    )(reordered.reshape(q, m), ranks)
    return out
